# baseline (speedup 1.0000x reference)
import jax
import jax.numpy as jnp
from jax.experimental import pallas as pl
from jax.experimental.pallas import tpu as pltpu

_EPS = 1e-12
_KP = 104
_NEG = -1e30
_NCHUNK = 1


def _nv_kernel(x_ref, w_ref, c_ref, out_ref):
    L = x_ref.shape[2]
    LC = L // _NCHUNK
    xf = x_ref[0]
    nrm2 = jnp.sum(xf * xf, axis=0, keepdims=True)
    inv_n = 1.0 / jnp.maximum(jnp.sqrt(nrm2), _EPS)
    ones8 = jnp.ones((8, LC), jnp.float32)
    agg = jnp.zeros((_KP, 72), jnp.float32)
    for ci in range(_NCHUNK):
        sl = slice(ci * LC, (ci + 1) * LC)
        xc = xf[:, sl]
        xa1 = jnp.concatenate([xc * inv_n[:, sl], ones8], axis=0)
        logits = jax.lax.dot_general(
            w_ref[...], xa1, (((1,), (0,)), ((), ())),
            preferred_element_type=jnp.float32)
        m = jnp.max(logits, axis=0, keepdims=True)
        e = jnp.exp(logits - m)
        s = jnp.sum(e, axis=0, keepdims=True)
        a = e.astype(jnp.bfloat16) * (1.0 / s).astype(jnp.bfloat16)
        xa2 = jnp.concatenate([xc, ones8], axis=0).astype(jnp.bfloat16)
        agg = agg + jax.lax.dot_general(
            a, xa2, (((1,), (1,)), ((), ())),
            preferred_element_type=jnp.float32)
    vlad = agg[:, 0:64] - c_ref[...] * agg[:, 64:65]
    rn = jnp.sum(vlad * vlad, axis=1, keepdims=True)
    vlad = vlad * (1.0 / jnp.maximum(jnp.sqrt(rn), _EPS))
    tot = jnp.sum(jnp.sum(vlad * vlad, axis=1, keepdims=True),
                  axis=0, keepdims=True)
    vlad = vlad * (1.0 / jnp.maximum(jnp.sqrt(tot), _EPS))
    out_ref[0] = vlad[:100, :]


def kernel(x, conv_w, conv_b, centroids):
    N, C, H, W = x.shape
    K = centroids.shape[0]
    L = H * W
    x3 = x.reshape(N, C, L)
    b_full = jnp.full((_KP,), _NEG, jnp.float32).at[:K].set(conv_b)
    b_hi = b_full.astype(jnp.bfloat16).astype(jnp.float32)
    w_aug = jnp.zeros((_KP, C + 8), jnp.float32)
    w_aug = w_aug.at[:K, :C].set(conv_w)
    w_aug = w_aug.at[:, C].set(b_hi)
    w_aug = w_aug.at[:, C + 1].set(b_full - b_hi)
    cent_p = jnp.zeros((_KP, C), jnp.float32).at[:K].set(centroids)
    out = pl.pallas_call(
        _nv_kernel,
        out_shape=jax.ShapeDtypeStruct((N, K, C), jnp.float32),
        grid=(N,),
        in_specs=[
            pl.BlockSpec((1, C, L), lambda i: (i, 0, 0)),
            pl.BlockSpec((_KP, C + 8), lambda i: (0, 0)),
            pl.BlockSpec((_KP, C), lambda i: (0, 0)),
        ],
        out_specs=pl.BlockSpec((1, K, C), lambda i: (i, 0, 0)),
        compiler_params=pltpu.CompilerParams(
            dimension_semantics=("arbitrary",),
            vmem_limit_bytes=56 * 1024 * 1024,
        ),
        name="netvlad_fused",
    )(x3, w_aug, cent_p)
    return out.reshape(N, K * C)


# speedup vs baseline: 1.1666x; 1.1666x over previous
import jax
import jax.numpy as jnp
from jax.experimental import pallas as pl
from jax.experimental.pallas import tpu as pltpu

_EPS = 1e-12
_KP = 104
_NEG = -1e30
_NCHUNK = 1
_NB = 2


def _nv_kernel(x_ref, w_ref, c_ref, out_ref):
    L = x_ref.shape[2]
    LC = L // _NCHUNK
    for ni in range(x_ref.shape[0]):
        _nv_one(x_ref, w_ref, c_ref, out_ref, ni, L, LC)


def _nv_one(x_ref, w_ref, c_ref, out_ref, ni, L, LC):
    xf = x_ref[ni]
    nrm2 = jnp.sum(xf * xf, axis=0, keepdims=True)
    inv_n = 1.0 / jnp.maximum(jnp.sqrt(nrm2), _EPS)
    ones8 = jnp.ones((8, LC), jnp.float32)
    agg = jnp.zeros((_KP, 72), jnp.float32)
    for ci in range(_NCHUNK):
        sl = slice(ci * LC, (ci + 1) * LC)
        xc = xf[:, sl]
        xa1 = jnp.concatenate([xc * inv_n[:, sl], ones8], axis=0)
        logits = jax.lax.dot_general(
            w_ref[...], xa1, (((1,), (0,)), ((), ())),
            preferred_element_type=jnp.float32)
        m = jnp.max(logits, axis=0, keepdims=True)
        e = jnp.exp(logits - m)
        s = jnp.sum(e, axis=0, keepdims=True)
        a = e.astype(jnp.bfloat16) * (1.0 / s).astype(jnp.bfloat16)
        xa2 = jnp.concatenate([xc, ones8], axis=0).astype(jnp.bfloat16)
        agg = agg + jax.lax.dot_general(
            a, xa2, (((1,), (1,)), ((), ())),
            preferred_element_type=jnp.float32)
    vlad = agg[:, 0:64] - c_ref[...] * agg[:, 64:65]
    rn = jnp.sum(vlad * vlad, axis=1, keepdims=True)
    vlad = vlad * (1.0 / jnp.maximum(jnp.sqrt(rn), _EPS))
    tot = jnp.sum(jnp.sum(vlad * vlad, axis=1, keepdims=True),
                  axis=0, keepdims=True)
    vlad = vlad * (1.0 / jnp.maximum(jnp.sqrt(tot), _EPS))
    out_ref[ni] = vlad[:100, :]


def kernel(x, conv_w, conv_b, centroids):
    N, C, H, W = x.shape
    K = centroids.shape[0]
    L = H * W
    x3 = x.reshape(N, C, L)
    b_full = jnp.full((_KP,), _NEG, jnp.float32).at[:K].set(conv_b)
    b_hi = b_full.astype(jnp.bfloat16).astype(jnp.float32)
    w_aug = jnp.zeros((_KP, C + 8), jnp.float32)
    w_aug = w_aug.at[:K, :C].set(conv_w)
    w_aug = w_aug.at[:, C].set(b_hi)
    w_aug = w_aug.at[:, C + 1].set(b_full - b_hi)
    cent_p = jnp.zeros((_KP, C), jnp.float32).at[:K].set(centroids)
    out = pl.pallas_call(
        _nv_kernel,
        out_shape=jax.ShapeDtypeStruct((N, K, C), jnp.float32),
        grid=(N // _NB,),
        in_specs=[
            pl.BlockSpec((_NB, C, L), lambda i: (i, 0, 0)),
            pl.BlockSpec((_KP, C + 8), lambda i: (0, 0)),
            pl.BlockSpec((_KP, C), lambda i: (0, 0)),
        ],
        out_specs=pl.BlockSpec((_NB, K, C), lambda i: (i, 0, 0)),
        compiler_params=pltpu.CompilerParams(
            dimension_semantics=("arbitrary",),
            vmem_limit_bytes=56 * 1024 * 1024,
        ),
        name="netvlad_fused",
    )(x3, w_aug, cent_p)
    return out.reshape(N, K * C)
